# baseline (speedup 1.0000x reference)
import dataclasses
import functools

import jax
import jax.numpy as jnp
from jax import lax
from jax.experimental import pallas as pl
from jax.experimental.pallas import tpu as pltpu
from jax.experimental.pallas import tpu_sc as plsc

BATCH = 16384
EMBED = 32
HID = 128

NC, NS = 2, 16
NW = NC * NS
BPW = BATCH // NW
ROWCHUNK = 128
NPASS = BPW // ROWCHUNK

_sc_mesh = plsc.VectorSubcoreMesh(core_axis_name="c", subcore_axis_name="s")

_sc_params = pltpu.CompilerParams()
if "needs_layout_passes" in pltpu.CompilerParams.__dataclass_fields__:
    _sc_params = dataclasses.replace(_sc_params, needs_layout_passes=False)


def _sc_gather4(user, item, t_um, t_im, t_ug, t_ig):
    f32 = jnp.float32
    out_t = tuple(
        jax.ShapeDtypeStruct((BATCH, EMBED), f32) for _ in range(4))

    @functools.partial(
        pl.kernel,
        mesh=_sc_mesh,
        out_type=out_t,
        scratch_types=[
            pltpu.VMEM((BPW,), jnp.int32),
            pltpu.VMEM((BPW,), jnp.int32),
            pltpu.VMEM((ROWCHUNK, EMBED), f32),
            pltpu.VMEM((ROWCHUNK, EMBED), f32),
            pltpu.VMEM((ROWCHUNK, EMBED), f32),
            pltpu.VMEM((ROWCHUNK, EMBED), f32),
            pltpu.SemaphoreType.DMA,
            pltpu.SemaphoreType.DMA,
        ],
        compiler_params=_sc_params,
    )
    def k(u_hbm, i_hbm, hum, him, hug, hig,
          oum, oim, oug, oig,
          uvm, ivm, bum, bim, bug, big, gsem, osem):
        wid = lax.axis_index("s") * NC + lax.axis_index("c")
        base = wid * BPW
        pltpu.sync_copy(u_hbm.at[pl.ds(base, BPW)], uvm)
        pltpu.sync_copy(i_hbm.at[pl.ds(base, BPW)], ivm)
        lanes = lax.broadcasted_iota(jnp.int32, (16,), 0)
        tabs = ((hum, uvm, bum, oum), (him, ivm, bim, oim),
                (hug, uvm, bug, oug), (hig, ivm, big, oig))
        for p in range(NPASS):
            if p:
                for _, _, buf, out in tabs:
                    pltpu.make_async_copy(
                        buf, out.at[pl.ds(base + (p - 1) * ROWCHUNK,
                                          ROWCHUNK)], osem).wait()

            @pl.loop(0, ROWCHUNK // 16)
            def _(b):
                uvec = uvm[pl.ds(p * ROWCHUNK + b * 16, 16)]
                ivec = ivm[pl.ds(p * ROWCHUNK + b * 16, 16)]
                for l in range(16):
                    u = jnp.max(jnp.where(lanes == l, uvec, -1))
                    v = jnp.max(jnp.where(lanes == l, ivec, -1))
                    for tbl, idx, buf, _ in (
                            (hum, u, bum, None), (him, v, bim, None),
                            (hug, u, bug, None), (hig, v, big, None)):
                        pltpu.make_async_copy(
                            tbl.at[pl.ds(idx, 1)],
                            buf.at[pl.ds(b * 16 + l, 1)],
                            gsem,
                        ).start()

            row0 = base + p * ROWCHUNK
            for tbl, _, buf, out in tabs:
                pltpu.make_async_copy(
                    tbl.at[pl.ds(0, ROWCHUNK)], buf, gsem).wait()
            for _, _, buf, out in tabs:
                pltpu.make_async_copy(
                    buf, out.at[pl.ds(row0, ROWCHUNK)], osem).start()
        for _, _, buf, out in tabs:
            pltpu.make_async_copy(
                buf, out.at[pl.ds(base + BPW - ROWCHUNK, ROWCHUNK)],
                osem).wait()

    return k(user, item, t_um, t_im, t_ug, t_ig)


_RB = 2048


def _tc_body(um_ref, im_ref, ug_ref, ig_ref, w1u_ref, w1i_ref, b1_ref,
             w2_ref, b2_ref, wm_ref, wg_ref, cb_ref, out_ref):
    hi = jax.lax.Precision.HIGHEST
    h = jnp.dot(um_ref[...], w1u_ref[...], precision=hi,
                preferred_element_type=jnp.float32)
    h += jnp.dot(im_ref[...], w1i_ref[...], precision=hi,
                 preferred_element_type=jnp.float32)
    h = jnp.maximum(h + b1_ref[...], 0.0)
    h = jnp.dot(h, w2_ref[...], precision=hi,
                preferred_element_type=jnp.float32)
    h = jnp.maximum(h + b2_ref[...], 0.0)
    mlp = jnp.sum(h * wm_ref[...], axis=1)
    gmf = jnp.sum(ug_ref[...] * ig_ref[...] * wg_ref[...], axis=1)
    out_ref[...] = mlp + gmf + cb_ref[0]


def _tc_dense(um, im, ug, ig, w1u, w1i, b1, w2t, b2, wm, wg, cb):
    grid = (BATCH // _RB,)
    row_spec = pl.BlockSpec((_RB, EMBED), lambda i: (i, 0))
    full = lambda shape: pl.BlockSpec(shape, lambda i: (0,) * len(shape))
    return pl.pallas_call(
        _tc_body,
        grid=grid,
        in_specs=[
            row_spec, row_spec, row_spec, row_spec,
            full((EMBED, HID)), full((EMBED, HID)), full((HID,)),
            full((HID, HID)), full((HID,)), full((HID,)), full((EMBED,)),
            full((1,)),
        ],
        out_specs=pl.BlockSpec((_RB,), lambda i: (i,)),
        out_shape=jax.ShapeDtypeStruct((BATCH,), jnp.float32),
    )(um, im, ug, ig, w1u, w1i, b1, w2t, b2, wm, wg, cb)


def kernel(user, item, user_emb_gmf, item_emb_gmf, user_emb_mlp, item_emb_mlp,
           W1, b1, W2, b2, Wm, bm, Wg, bg):
    user = user.astype(jnp.int32)
    item = item.astype(jnp.int32)
    um, im, ug, ig = _sc_gather4(user, item, user_emb_mlp, item_emb_mlp,
                                 user_emb_gmf, item_emb_gmf)
    w1u = W1[:, :EMBED].T
    w1i = W1[:, EMBED:].T
    w2t = W2.T
    wm = Wm[0]
    wg = Wg[0]
    cb = (bm + bg).reshape(1)
    return _tc_dense(um, im, ug, ig, w1u, w1i, b1, w2t, b2, wm, wg, cb)
